# baseline (device time: 1480931 ns/iter reference)
import jax
import jax.numpy as jnp
from jax import lax
from jax.experimental import pallas as pl
from jax.experimental.pallas import tpu as pltpu

N_DEV = 16
M_BLK = 256
N_OUT = 8192
CB = 2048


def kernel(x, w_mat):
    x_hi = x.astype(jnp.bfloat16)
    x_lo = (x - x_hi.astype(jnp.float32)).astype(jnp.bfloat16)
    w_hi = w_mat.astype(jnp.bfloat16)
    w_lo = (w_mat - w_hi.astype(jnp.float32)).astype(jnp.bfloat16)
    x_cat = jnp.concatenate([x_hi, x_lo, x_hi], axis=1)
    w_cat = jnp.concatenate([w_hi, w_hi, w_lo], axis=0)

    def body(x_ref, w_ref, out_ref, comm, send_sems, recv_sems,
             credit_sem, amax_buf, ax_send, ax_recv):
        d = lax.axis_index("i")
        left = lax.rem(d + N_DEV - 1, N_DEV)
        right = lax.rem(d + 1, N_DEV)

        barrier = pltpu.get_barrier_semaphore()
        for nbr in (left, right):
            pl.semaphore_signal(barrier, inc=1, device_id=(nbr,),
                                device_id_type=pl.DeviceIdType.MESH)
        pl.semaphore_wait(barrier, 2)

        def mm(c, j0):
            xb = x_ref[pl.ds(c * M_BLK, M_BLK), :]
            wb = w_ref[:, pl.ds(j0, CB)]
            return lax.dot_general(
                xb, wb, (((1,), (0,)), ((), ())),
                preferred_element_type=jnp.float32)

        for s in range(N_DEV - 1):
            c = lax.rem(d + 2 * N_DEV - 1 - s, N_DEV)
            slot = s % 2
            for j0 in range(0, N_OUT, CB):
                if s == 0:
                    comm[slot, :, pl.ds(j0, CB)] = mm(c, j0)
                else:
                    comm[slot, :, pl.ds(j0, CB)] = (
                        comm[slot, :, pl.ds(j0, CB)] + mm(c, j0))
            if s >= 1:
                pl.semaphore_wait(credit_sem, 1)
            rdma = pltpu.make_async_remote_copy(
                src_ref=comm.at[slot],
                dst_ref=comm.at[(s + 1) % 2],
                send_sem=send_sems.at[slot],
                recv_sem=recv_sems.at[(s + 1) % 2],
                device_id=(right,),
                device_id_type=pl.DeviceIdType.MESH,
            )
            rdma.start()
            rdma.wait()
            if s <= N_DEV - 3:
                pl.semaphore_signal(credit_sem, inc=1, device_id=(left,),
                                    device_id_type=pl.DeviceIdType.MESH)

        fslot = (N_DEV - 1) % 2
        for j0 in range(0, N_OUT, CB):
            out_ref[:, pl.ds(j0, CB)] = jnp.maximum(
                comm[fslot, :, pl.ds(j0, CB)] + mm(d, j0), 0.0)

        local_amax = jnp.max(out_ref[...])
        amax_buf[0, :] = jnp.full((128,), local_amax, jnp.float32)
        rdmas = []
        for r in range(1, N_DEV):
            tgt = lax.rem(d + r, N_DEV)
            rd = pltpu.make_async_remote_copy(
                src_ref=amax_buf.at[0],
                dst_ref=amax_buf.at[r],
                send_sem=ax_send.at[r],
                recv_sem=ax_recv.at[r],
                device_id=(tgt,),
                device_id_type=pl.DeviceIdType.MESH,
            )
            rd.start()
            rdmas.append(rd)
        for rd in rdmas:
            rd.wait_send()
        for rd in rdmas:
            rd.wait_recv()
        gmax = jnp.max(amax_buf[...])

        scale = gmax / 448.0
        inv_scale = 448.0 / gmax
        for j0 in range(0, N_OUT, CB):
            y = out_ref[:, pl.ds(j0, CB)]
            q = jnp.minimum(y * inv_scale, 448.0)
            q8 = q.astype(jnp.float8_e4m3fn)
            out_ref[:, pl.ds(j0, CB)] = q8.astype(jnp.float32) * scale

    return pl.pallas_call(
        body,
        out_shape=jax.ShapeDtypeStruct((M_BLK, N_OUT), jnp.float32),
        in_specs=[
            pl.BlockSpec(memory_space=pltpu.VMEM),
            pl.BlockSpec(memory_space=pltpu.VMEM),
        ],
        out_specs=pl.BlockSpec(memory_space=pltpu.VMEM),
        scratch_shapes=[
            pltpu.VMEM((2, M_BLK, N_OUT), jnp.float32),
            pltpu.SemaphoreType.DMA((2,)),
            pltpu.SemaphoreType.DMA((2,)),
            pltpu.SemaphoreType.REGULAR,
            pltpu.VMEM((N_DEV, 128), jnp.float32),
            pltpu.SemaphoreType.DMA((N_DEV,)),
            pltpu.SemaphoreType.DMA((N_DEV,)),
        ],
        compiler_params=pltpu.CompilerParams(collective_id=0),
    )(x_cat, w_cat)


# device time: 811402 ns/iter; 1.8252x vs baseline; 1.8252x over previous
import jax
import jax.numpy as jnp
from jax import lax
from jax.experimental import pallas as pl
from jax.experimental.pallas import tpu as pltpu

N_DEV = 16
M_BLK = 256
N_OUT = 8192
HALF = N_OUT // 2
CB = 2048


def kernel(x, w_mat):
    x_hi = x.astype(jnp.bfloat16)
    x_lo = (x - x_hi.astype(jnp.float32)).astype(jnp.bfloat16)
    w_hi = w_mat.astype(jnp.bfloat16)
    w_lo = (w_mat - w_hi.astype(jnp.float32)).astype(jnp.bfloat16)
    x_cat = jnp.concatenate([x_hi, x_lo, x_hi], axis=1)
    w_cat = jnp.concatenate([w_hi, w_hi, w_lo], axis=0)

    def body(x_ref, w_ref, out_ref, comm_l, comm_r,
             send_l, recv_l, send_r, recv_r, credit_l, credit_r,
             amax_buf, ax_send, ax_recv):
        d = lax.axis_index("i")
        left = lax.rem(d + N_DEV - 1, N_DEV)
        right = lax.rem(d + 1, N_DEV)

        barrier = pltpu.get_barrier_semaphore()
        for nbr in (left, right):
            pl.semaphore_signal(barrier, inc=1, device_id=(nbr,),
                                device_id_type=pl.DeviceIdType.MESH)
        pl.semaphore_wait(barrier, 2)

        def mm(c, j0):
            xb = x_ref[pl.ds(c * M_BLK, M_BLK), :]
            wb = w_ref[:, pl.ds(j0, CB)]
            return lax.dot_general(
                xb, wb, (((1,), (0,)), ((), ())),
                preferred_element_type=jnp.float32)

        def chunk_l(s):
            return lax.rem(d + 2 * N_DEV - 1 - s, N_DEV)

        def chunk_r(s):
            return lax.rem(d + 1 + s, N_DEV)

        for j0 in range(0, HALF, CB):
            comm_l[0, :, pl.ds(j0, CB)] = mm(chunk_l(0), j0)
            comm_r[0, :, pl.ds(j0, CB)] = mm(chunk_r(0), HALF + j0)

        for s in range(N_DEV - 1):
            slot, nslot = s % 2, (s + 1) % 2
            if s >= 1:
                pl.semaphore_wait(credit_l, 1)
                pl.semaphore_wait(credit_r, 1)
            rdma_l = pltpu.make_async_remote_copy(
                src_ref=comm_l.at[slot], dst_ref=comm_l.at[nslot],
                send_sem=send_l.at[slot], recv_sem=recv_l.at[nslot],
                device_id=(right,), device_id_type=pl.DeviceIdType.MESH)
            rdma_r = pltpu.make_async_remote_copy(
                src_ref=comm_r.at[slot], dst_ref=comm_r.at[nslot],
                send_sem=send_r.at[slot], recv_sem=recv_r.at[nslot],
                device_id=(left,), device_id_type=pl.DeviceIdType.MESH)
            rdma_l.start()
            rdma_r.start()
            for j0 in range(0, HALF, CB):
                out_ref[:, pl.ds(j0, CB)] = mm(chunk_l(s + 1), j0)
                out_ref[:, pl.ds(HALF + j0, CB)] = mm(chunk_r(s + 1), HALF + j0)
            rdma_l.wait()
            rdma_r.wait()
            for j0 in range(0, HALF, CB):
                acc_l = comm_l[nslot, :, pl.ds(j0, CB)] + out_ref[:, pl.ds(j0, CB)]
                acc_r = comm_r[nslot, :, pl.ds(j0, CB)] + out_ref[:, pl.ds(HALF + j0, CB)]
                if s == N_DEV - 2:
                    out_ref[:, pl.ds(j0, CB)] = jnp.maximum(acc_l, 0.0)
                    out_ref[:, pl.ds(HALF + j0, CB)] = jnp.maximum(acc_r, 0.0)
                else:
                    comm_l[nslot, :, pl.ds(j0, CB)] = acc_l
                    comm_r[nslot, :, pl.ds(j0, CB)] = acc_r
            if s <= N_DEV - 3:
                pl.semaphore_signal(credit_l, inc=1, device_id=(left,),
                                    device_id_type=pl.DeviceIdType.MESH)
                pl.semaphore_signal(credit_r, inc=1, device_id=(right,),
                                    device_id_type=pl.DeviceIdType.MESH)

        local_amax = jnp.max(out_ref[...])
        amax_buf[0, :] = jnp.full((128,), local_amax, jnp.float32)
        rdmas = []
        for r in range(1, N_DEV):
            tgt = lax.rem(d + r, N_DEV)
            rd = pltpu.make_async_remote_copy(
                src_ref=amax_buf.at[0], dst_ref=amax_buf.at[r],
                send_sem=ax_send.at[r], recv_sem=ax_recv.at[r],
                device_id=(tgt,), device_id_type=pl.DeviceIdType.MESH)
            rd.start()
            rdmas.append(rd)
        for rd in rdmas:
            rd.wait_send()
        for rd in rdmas:
            rd.wait_recv()
        gmax = jnp.max(amax_buf[...])

        scale = gmax / 448.0
        inv_scale = 448.0 / gmax
        for j0 in range(0, N_OUT, CB):
            y = out_ref[:, pl.ds(j0, CB)]
            q = jnp.minimum(y * inv_scale, 448.0)
            q8 = q.astype(jnp.float8_e4m3fn)
            out_ref[:, pl.ds(j0, CB)] = q8.astype(jnp.float32) * scale

    return pl.pallas_call(
        body,
        out_shape=jax.ShapeDtypeStruct((M_BLK, N_OUT), jnp.float32),
        in_specs=[
            pl.BlockSpec(memory_space=pltpu.VMEM),
            pl.BlockSpec(memory_space=pltpu.VMEM),
        ],
        out_specs=pl.BlockSpec(memory_space=pltpu.VMEM),
        scratch_shapes=[
            pltpu.VMEM((2, M_BLK, HALF), jnp.float32),
            pltpu.VMEM((2, M_BLK, HALF), jnp.float32),
            pltpu.SemaphoreType.DMA((2,)),
            pltpu.SemaphoreType.DMA((2,)),
            pltpu.SemaphoreType.DMA((2,)),
            pltpu.SemaphoreType.DMA((2,)),
            pltpu.SemaphoreType.REGULAR,
            pltpu.SemaphoreType.REGULAR,
            pltpu.VMEM((N_DEV, 128), jnp.float32),
            pltpu.SemaphoreType.DMA((N_DEV,)),
            pltpu.SemaphoreType.DMA((N_DEV,)),
        ],
        compiler_params=pltpu.CompilerParams(collective_id=0),
    )(x_cat, w_cat)


# device time: 722504 ns/iter; 2.0497x vs baseline; 1.1230x over previous
import jax
import jax.numpy as jnp
from jax import lax
from jax.experimental import pallas as pl
from jax.experimental.pallas import tpu as pltpu

N_DEV = 16
M_BLK = 256
N_OUT = 8192
HALF = N_OUT // 2
QTR = N_OUT // 4


def kernel(x, w_mat):
    x_hi = x.astype(jnp.bfloat16)
    x_lo = (x - x_hi.astype(jnp.float32)).astype(jnp.bfloat16)
    w_hi = w_mat.astype(jnp.bfloat16)
    w_lo = (w_mat - w_hi.astype(jnp.float32)).astype(jnp.bfloat16)
    x_cat = jnp.concatenate([x_hi, x_lo, x_hi], axis=1)
    w_cat = jnp.concatenate([w_hi, w_hi, w_lo], axis=0)

    def body(x_ref, w_ref, out_ref, comm_l, comm_r,
             send_l, recv_l, send_r, recv_r,
             credit_la, credit_lb, credit_ra, credit_rb,
             amax_buf, ax_send, ax_recv):
        d = lax.axis_index("i")
        left = lax.rem(d + N_DEV - 1, N_DEV)
        right = lax.rem(d + 1, N_DEV)

        barrier = pltpu.get_barrier_semaphore()
        for nbr in (left, right):
            pl.semaphore_signal(barrier, inc=1, device_id=(nbr,),
                                device_id_type=pl.DeviceIdType.MESH)
        pl.semaphore_wait(barrier, 2)

        def mm(c, j0):
            xb = x_ref[pl.ds(c * M_BLK, M_BLK), :]
            wb = w_ref[:, pl.ds(j0, QTR)]
            return lax.dot_general(
                xb, wb, (((1,), (0,)), ((), ())),
                preferred_element_type=jnp.float32)

        def chunk_l(s):
            return lax.rem(d + 2 * N_DEV - 1 - s, N_DEV)

        def chunk_r(s):
            return lax.rem(d + 1 + s, N_DEV)

        rings = [
            ("LA", comm_l, 0, send_l, recv_l, right, credit_la, left, 0),
            ("RA", comm_r, 0, send_r, recv_r, left, credit_ra, right, HALF),
            ("LB", comm_l, 1, send_l, recv_l, right, credit_lb, left, QTR),
            ("RB", comm_r, 1, send_r, recv_r, left, credit_rb, right, HALF + QTR),
        ]
        rds = {name: [] for name, *_ in rings}

        def make_rdma(ring, s):
            name, comm, q, ssem, rsem, peer, _cin, _cpeer, _j0 = ring
            slot, nslot = s % 2, (s + 1) % 2
            return pltpu.make_async_remote_copy(
                src_ref=comm.at[q, slot], dst_ref=comm.at[q, nslot],
                send_sem=ssem.at[q, slot], recv_sem=rsem.at[q, nslot],
                device_id=(peer,), device_id_type=pl.DeviceIdType.MESH)

        def stage_col(ring):
            return ring[8]

        def fold(ring, s, final):
            name, comm, q, _ssem, _rsem, _peer, _cin, _cpeer, j0 = ring
            nslot = (s + 1) % 2
            if s >= 1:
                rds[name][s - 1].wait_send()
            acc = comm[q, nslot, :, pl.ds(0, QTR)] + out_ref[:, pl.ds(j0, QTR)]
            if final:
                out_ref[:, pl.ds(j0, QTR)] = jnp.maximum(acc, 0.0)
            else:
                comm[q, nslot, :, pl.ds(0, QTR)] = acc

        for ring in rings:
            _, comm, q, *_rest = ring
            cj = chunk_l(0) if ring[0][0] == "L" else chunk_r(0)
            comm[q, 0, :, pl.ds(0, QTR)] = mm(cj, stage_col(ring))

        for s in range(N_DEV - 1):
            for pair in (rings[0:2], rings[2:4]):
                for ring in pair:
                    name, comm, q, ssem, rsem, peer, cin, cpeer, j0 = ring
                    if s >= 1:
                        rds[name][s - 1].wait_recv()
                        fold(ring, s - 1, final=False)
                        pl.semaphore_signal(
                            cin, inc=1, device_id=(cpeer,),
                            device_id_type=pl.DeviceIdType.MESH)
                        pl.semaphore_wait(cin, 1)
                    rd = make_rdma(ring, s)
                    rds[name].append(rd)
                    rd.start()
            out_ref[:, pl.ds(0, QTR)] = mm(chunk_l(s + 1), 0)
            out_ref[:, pl.ds(QTR, QTR)] = mm(chunk_l(s + 1), QTR)
            out_ref[:, pl.ds(HALF, QTR)] = mm(chunk_r(s + 1), HALF)
            out_ref[:, pl.ds(HALF + QTR, QTR)] = mm(chunk_r(s + 1), HALF + QTR)

        for ring in rings:
            name = ring[0]
            rds[name][N_DEV - 2].wait_recv()
            fold(ring, N_DEV - 2, final=True)
        for ring in rings:
            rds[ring[0]][N_DEV - 2].wait_send()

        local_amax = jnp.max(out_ref[...])
        amax_buf[0, :] = jnp.full((128,), local_amax, jnp.float32)
        rdmas = []
        for r in range(1, N_DEV):
            tgt = lax.rem(d + r, N_DEV)
            rd = pltpu.make_async_remote_copy(
                src_ref=amax_buf.at[0], dst_ref=amax_buf.at[r],
                send_sem=ax_send.at[r], recv_sem=ax_recv.at[r],
                device_id=(tgt,), device_id_type=pl.DeviceIdType.MESH)
            rd.start()
            rdmas.append(rd)
        for rd in rdmas:
            rd.wait_send()
        for rd in rdmas:
            rd.wait_recv()
        gmax = jnp.max(amax_buf[...])

        scale = gmax / 448.0
        inv_scale = 448.0 / gmax
        for j0 in range(0, N_OUT, HALF):
            y = out_ref[:, pl.ds(j0, HALF)]
            q = jnp.minimum(y * inv_scale, 448.0)
            q8 = q.astype(jnp.float8_e4m3fn)
            out_ref[:, pl.ds(j0, HALF)] = q8.astype(jnp.float32) * scale

    return pl.pallas_call(
        body,
        out_shape=jax.ShapeDtypeStruct((M_BLK, N_OUT), jnp.float32),
        in_specs=[
            pl.BlockSpec(memory_space=pltpu.VMEM),
            pl.BlockSpec(memory_space=pltpu.VMEM),
        ],
        out_specs=pl.BlockSpec(memory_space=pltpu.VMEM),
        scratch_shapes=[
            pltpu.VMEM((2, 2, M_BLK, QTR), jnp.float32),
            pltpu.VMEM((2, 2, M_BLK, QTR), jnp.float32),
            pltpu.SemaphoreType.DMA((2, 2)),
            pltpu.SemaphoreType.DMA((2, 2)),
            pltpu.SemaphoreType.DMA((2, 2)),
            pltpu.SemaphoreType.DMA((2, 2)),
            pltpu.SemaphoreType.REGULAR,
            pltpu.SemaphoreType.REGULAR,
            pltpu.SemaphoreType.REGULAR,
            pltpu.SemaphoreType.REGULAR,
            pltpu.VMEM((N_DEV, 128), jnp.float32),
            pltpu.SemaphoreType.DMA((N_DEV,)),
            pltpu.SemaphoreType.DMA((N_DEV,)),
        ],
        compiler_params=pltpu.CompilerParams(collective_id=0),
    )(x_cat, w_cat)


# device time: 704036 ns/iter; 2.1035x vs baseline; 1.0262x over previous
import jax
import jax.numpy as jnp
from jax import lax
from jax.experimental import pallas as pl
from jax.experimental.pallas import tpu as pltpu

N_DEV = 16
M_BLK = 256
N_OUT = 8192
HALF = N_OUT // 2
QTR = N_OUT // 4


def kernel(x, w_mat):
    x_hi = x.astype(jnp.bfloat16)
    x_lo = (x - x_hi.astype(jnp.float32)).astype(jnp.bfloat16)
    w_hi = w_mat.astype(jnp.bfloat16)
    w_lo = (w_mat - w_hi.astype(jnp.float32)).astype(jnp.bfloat16)

    def body(xh_ref, xl_ref, wh_ref, wl_ref, out_ref, comm_l, comm_r,
             send_l, recv_l, send_r, recv_r,
             credit_la, credit_lb, credit_ra, credit_rb,
             amax_buf, ax_send, ax_recv):
        d = lax.axis_index("i")
        left = lax.rem(d + N_DEV - 1, N_DEV)
        right = lax.rem(d + 1, N_DEV)

        def mm(c, j0):
            dn = (((1,), (0,)), ((), ()))
            xh = xh_ref[pl.ds(c * M_BLK, M_BLK), :]
            xl = xl_ref[pl.ds(c * M_BLK, M_BLK), :]
            wh = wh_ref[:, pl.ds(j0, QTR)]
            wl = wl_ref[:, pl.ds(j0, QTR)]
            return (lax.dot_general(xh, wh, dn, preferred_element_type=jnp.float32)
                    + lax.dot_general(xl, wh, dn, preferred_element_type=jnp.float32)
                    + lax.dot_general(xh, wl, dn, preferred_element_type=jnp.float32))

        def chunk_l(s):
            return lax.rem(d + 2 * N_DEV - 1 - s, N_DEV)

        def chunk_r(s):
            return lax.rem(d + 1 + s, N_DEV)

        rings = [
            ("LA", comm_l, 0, send_l, recv_l, right, credit_la, left, 0),
            ("RA", comm_r, 0, send_r, recv_r, left, credit_ra, right, HALF),
            ("LB", comm_l, 1, send_l, recv_l, right, credit_lb, left, QTR),
            ("RB", comm_r, 1, send_r, recv_r, left, credit_rb, right, HALF + QTR),
        ]
        rds = {name: [] for name, *_ in rings}

        def make_rdma(ring, s):
            name, comm, q, ssem, rsem, peer, _cin, _cpeer, _j0 = ring
            slot, nslot = s % 2, (s + 1) % 2
            return pltpu.make_async_remote_copy(
                src_ref=comm.at[q, slot], dst_ref=comm.at[q, nslot],
                send_sem=ssem.at[q, slot], recv_sem=rsem.at[q, nslot],
                device_id=(peer,), device_id_type=pl.DeviceIdType.MESH)

        def stage_col(ring):
            return ring[8]

        def fold(ring, s, final):
            name, comm, q, _ssem, _rsem, _peer, _cin, _cpeer, j0 = ring
            nslot = (s + 1) % 2
            if s >= 1:
                rds[name][s - 1].wait_send()
            acc = comm[q, nslot, :, pl.ds(0, QTR)] + out_ref[:, pl.ds(j0, QTR)]
            if final:
                out_ref[:, pl.ds(j0, QTR)] = jnp.maximum(acc, 0.0)
            else:
                comm[q, nslot, :, pl.ds(0, QTR)] = acc

        for ring in rings:
            _, comm, q, *_rest = ring
            cj = chunk_l(0) if ring[0][0] == "L" else chunk_r(0)
            comm[q, 0, :, pl.ds(0, QTR)] = mm(cj, stage_col(ring))

        barrier = pltpu.get_barrier_semaphore()
        for nbr in (left, right):
            pl.semaphore_signal(barrier, inc=1, device_id=(nbr,),
                                device_id_type=pl.DeviceIdType.MESH)
        pl.semaphore_wait(barrier, 2)

        for s in range(N_DEV - 1):
            for pair in (rings[0:2], rings[2:4]):
                for ring in pair:
                    name, comm, q, ssem, rsem, peer, cin, cpeer, j0 = ring
                    if s >= 1:
                        rds[name][s - 1].wait_recv()
                        fold(ring, s - 1, final=False)
                        pl.semaphore_signal(
                            cin, inc=1, device_id=(cpeer,),
                            device_id_type=pl.DeviceIdType.MESH)
                        pl.semaphore_wait(cin, 1)
                    rd = make_rdma(ring, s)
                    rds[name].append(rd)
                    rd.start()
            out_ref[:, pl.ds(0, QTR)] = mm(chunk_l(s + 1), 0)
            out_ref[:, pl.ds(QTR, QTR)] = mm(chunk_l(s + 1), QTR)
            out_ref[:, pl.ds(HALF, QTR)] = mm(chunk_r(s + 1), HALF)
            out_ref[:, pl.ds(HALF + QTR, QTR)] = mm(chunk_r(s + 1), HALF + QTR)

        for ring in rings:
            name = ring[0]
            rds[name][N_DEV - 2].wait_recv()
            fold(ring, N_DEV - 2, final=True)
        for ring in rings:
            rds[ring[0]][N_DEV - 2].wait_send()

        local_amax = jnp.max(out_ref[...])
        amax_buf[0, :] = jnp.full((128,), local_amax, jnp.float32)
        rdmas = []
        for r in range(1, N_DEV):
            tgt = lax.rem(d + r, N_DEV)
            rd = pltpu.make_async_remote_copy(
                src_ref=amax_buf.at[0], dst_ref=amax_buf.at[r],
                send_sem=ax_send.at[r], recv_sem=ax_recv.at[r],
                device_id=(tgt,), device_id_type=pl.DeviceIdType.MESH)
            rd.start()
            rdmas.append(rd)
        for rd in rdmas:
            rd.wait_send()
        for rd in rdmas:
            rd.wait_recv()
        gmax = jnp.max(amax_buf[...])

        scale = gmax / 448.0
        inv_scale = 448.0 / gmax
        for j0 in range(0, N_OUT, HALF):
            y = out_ref[:, pl.ds(j0, HALF)]
            q = jnp.minimum(y * inv_scale, 448.0)
            q8 = q.astype(jnp.float8_e4m3fn)
            out_ref[:, pl.ds(j0, HALF)] = q8.astype(jnp.float32) * scale

    return pl.pallas_call(
        body,
        out_shape=jax.ShapeDtypeStruct((M_BLK, N_OUT), jnp.float32),
        in_specs=[
            pl.BlockSpec(memory_space=pltpu.VMEM),
            pl.BlockSpec(memory_space=pltpu.VMEM),
            pl.BlockSpec(memory_space=pltpu.VMEM),
            pl.BlockSpec(memory_space=pltpu.VMEM),
        ],
        out_specs=pl.BlockSpec(memory_space=pltpu.VMEM),
        scratch_shapes=[
            pltpu.VMEM((2, 2, M_BLK, QTR), jnp.float32),
            pltpu.VMEM((2, 2, M_BLK, QTR), jnp.float32),
            pltpu.SemaphoreType.DMA((2, 2)),
            pltpu.SemaphoreType.DMA((2, 2)),
            pltpu.SemaphoreType.DMA((2, 2)),
            pltpu.SemaphoreType.DMA((2, 2)),
            pltpu.SemaphoreType.REGULAR,
            pltpu.SemaphoreType.REGULAR,
            pltpu.SemaphoreType.REGULAR,
            pltpu.SemaphoreType.REGULAR,
            pltpu.VMEM((N_DEV, 128), jnp.float32),
            pltpu.SemaphoreType.DMA((N_DEV,)),
            pltpu.SemaphoreType.DMA((N_DEV,)),
        ],
        compiler_params=pltpu.CompilerParams(collective_id=0),
    )(x_hi, x_lo, w_hi, w_lo)
